# Initial kernel scaffold; baseline (speedup 1.0000x reference)
import jax
import jax.numpy as jnp
from jax.experimental import pallas as pl


def kernel(input, noise):
    raise NotImplementedError("write your pallas kernel here")


# speedup vs baseline: 5.9731x; 5.9731x over previous
import jax
import jax.numpy as jnp
from jax.experimental import pallas as pl
from jax.experimental.pallas import tpu as pltpu

_N = 4096
_ROWS = 256


def _phi_block_kernel(shift_ref, x_ref, o_ref):
    x = x_ref[...]
    shift = shift_ref[0, 0]
    col = jax.lax.broadcasted_iota(jnp.int32, x.shape, 1)
    mask = (col % 3) == 1
    v = x + shift
    v = jnp.where(v > 1.0, v - 2.0, v)
    o_ref[...] = jnp.where(mask, v, x)


def kernel(input, noise):
    shift = (noise * 2.0 - 1.0).reshape(1, 1)
    grid = (_N // _ROWS,)
    return pl.pallas_call(
        _phi_block_kernel,
        grid=grid,
        in_specs=[
            pl.BlockSpec(memory_space=pltpu.SMEM),
            pl.BlockSpec((_ROWS, _N), lambda i: (i, 0)),
        ],
        out_specs=pl.BlockSpec((_ROWS, _N), lambda i: (i, 0)),
        out_shape=jax.ShapeDtypeStruct((_N, _N), jnp.float32),
        compiler_params=pltpu.CompilerParams(
            dimension_semantics=("parallel",),
        ),
    )(shift, input)
